# baseline (speedup 1.0000x reference)
import functools

import jax
import jax.numpy as jnp
from jax import lax
from jax.experimental import pallas as pl
from jax.experimental.pallas import tpu as pltpu
from jax.experimental.pallas import tpu_sc as plsc

B, A, D, KM1, K_TOP = 64, 4096, 256, 64, 64
B_TILE = 4

_NC, _NS, _L = 2, 16, 16
_NW = _NC * _NS
_RPW = B // _NW
_NV = A // _L


def _score_body(x_ref, w_ref, b_ref, o_ref):
    x = x_ref[...].reshape(B_TILE * A, D)
    logits_t = jax.lax.dot_general(
        w_ref[...], x, (((0,), (1,)), ((), ())),
        preferred_element_type=jnp.float32)
    s = jax.nn.sigmoid(logits_t + b_ref[...])
    o_ref[...] = (s.sum(axis=0) * (1.0 / KM1) - 0.5).reshape(B_TILE, 1, A)


def _sc_select_body(scores_hbm, out_hbm, row_v, bits_v, act_v):
    wid = lax.axis_index("s") * _NC + lax.axis_index("c")
    for rr in range(_RPW):
        r = wid * _RPW + rr
        pltpu.sync_copy(scores_hbm.at[r, 0], row_v)

        def p1(v, carry):
            sl = pl.ds(v * _L, _L)
            bits_v[sl] = (lax.bitcast_convert_type(row_v[sl], jnp.int32)
                          & 0x7FFFFFFF)
            return carry
        lax.fori_loop(0, _NV, p1, jnp.int32(0))

        def citer(_, lohi):
            lo, hi = lohi
            mid = lo + ((hi - lo) >> 1)

            def inner(v, cnt):
                m = bits_v[pl.ds(v * _L, _L)] >= mid
                return cnt + plsc.all_reduce_population_count(m)

            cnt = lax.fori_loop(0, _NV, inner, jnp.zeros((_L,), jnp.int32))
            ge = cnt >= K_TOP
            return jnp.where(ge, mid, lo), jnp.where(ge, hi, mid)

        t, _hi = lax.fori_loop(
            0, 31, citer,
            (jnp.zeros((_L,), jnp.int32),
             jnp.full((_L,), 0x7F800000, jnp.int32)))

        def p3(v, c):
            n, z = c
            b = bits_v[pl.ds(v * _L, _L)]
            m = b > t
            n = n + plsc.all_reduce_population_count(m)
            z = z + jnp.where(m, lax.bitcast_convert_type(b, jnp.float32), 0.0)
            return n, z

        n_gt, z_vec = lax.fori_loop(
            0, _NV, p3,
            (jnp.zeros((_L,), jnp.int32), jnp.zeros((_L,), jnp.float32)))
        need = K_TOP - n_gt
        t_f = lax.bitcast_convert_type(t, jnp.float32)
        z_sum = jnp.broadcast_to(jnp.sum(z_vec), (_L,))
        z = z_sum + need.astype(jnp.float32) * t_f
        inv = 1.0 / (z + 1e-8)

        def p4(v, carry_eq):
            sl = pl.ds(v * _L, _L)
            b = bits_v[sl]
            s = row_v[sl]
            gt = b > t
            eq = b == t
            eqi = eq.astype(jnp.int32)
            rank = carry_eq + (plsc.cumsum(eqi) - eqi)
            mask = gt | (eq & (rank < need))
            act_v[sl] = jnp.where(mask, s * inv, 0.0)
            return carry_eq + plsc.all_reduce_population_count(eq)

        lax.fori_loop(0, _NV, p4, jnp.zeros((_L,), jnp.int32))
        pltpu.sync_copy(act_v, out_hbm.at[r])


@functools.cache
def _make_sc_select():
    return pl.kernel(
        _sc_select_body,
        mesh=plsc.VectorSubcoreMesh(core_axis_name="c", subcore_axis_name="s"),
        out_type=jax.ShapeDtypeStruct((B, A), jnp.float32),
        scratch_types=[
            pltpu.VMEM((A,), jnp.float32),
            pltpu.VMEM((A,), jnp.int32),
            pltpu.VMEM((A,), jnp.float32),
        ],
        compiler_params=pltpu.CompilerParams(needs_layout_passes=False),
    )


@jax.jit
def kernel(signal_features, W, b):
    scores = pl.pallas_call(
        _score_body,
        grid=(B // B_TILE,),
        in_specs=[
            pl.BlockSpec((B_TILE, A, D), lambda i: (i, 0, 0)),
            pl.BlockSpec((D, KM1), lambda i: (0, 0)),
            pl.BlockSpec((KM1, 1), lambda i: (0, 0)),
        ],
        out_specs=pl.BlockSpec((B_TILE, 1, A), lambda i: (i, 0, 0)),
        out_shape=jax.ShapeDtypeStruct((B, 1, A), jnp.float32),
    )(signal_features, W, b.reshape(KM1, 1))
    return _make_sc_select()(scores)


# speedup vs baseline: 1.5233x; 1.5233x over previous
import functools

import jax
import jax.numpy as jnp
from jax import lax
from jax.experimental import pallas as pl
from jax.experimental.pallas import tpu as pltpu
from jax.experimental.pallas import tpu_sc as plsc

B, A, D, KM1, K_TOP = 64, 4096, 256, 64, 64
B_TILE = 4

_NC, _NS, _L = 2, 16, 16
_NW = _NC * _NS
_RPW = B // _NW
_NV = A // _L


def _score_body(x_ref, w_ref, b_ref, o_ref):
    x = x_ref[...].reshape(B_TILE * A, D)
    logits_t = jax.lax.dot_general(
        w_ref[...], x, (((0,), (1,)), ((), ())),
        preferred_element_type=jnp.float32)
    s = jax.nn.sigmoid(logits_t + b_ref[...])
    o_ref[...] = (s.sum(axis=0) * (1.0 / KM1) - 0.5).reshape(B_TILE, 1, A)


def _sc_select_body(scores_hbm, out_hbm, row_v, bits_v, act_v):
    wid = lax.axis_index("s") * _NC + lax.axis_index("c")
    for rr in range(_RPW):
        r = wid * _RPW + rr
        pltpu.sync_copy(scores_hbm.at[r, 0], row_v)

        def p1(v, carry):
            sl = pl.ds(v * _L, _L)
            bits_v[sl] = (lax.bitcast_convert_type(row_v[sl], jnp.int32)
                          & 0x7FFFFFFF)
            return carry
        lax.fori_loop(0, _NV, p1, jnp.int32(0))

        _UNROLL = 8

        def citer(_, lohi):
            lo, hi = lohi
            mid = lo + ((hi - lo) >> 1)

            def inner(v, cnt):
                c0, c1 = cnt
                for u in range(_UNROLL):
                    m = bits_v[pl.ds((v * _UNROLL + u) * _L, _L)] >= mid
                    p = plsc.all_reduce_population_count(m)
                    if u % 2 == 0:
                        c0 = c0 + p
                    else:
                        c1 = c1 + p
                return c0, c1

            c0, c1 = lax.fori_loop(
                0, _NV // _UNROLL, inner,
                (jnp.zeros((_L,), jnp.int32), jnp.zeros((_L,), jnp.int32)))
            ge = (c0 + c1) >= K_TOP
            return jnp.where(ge, mid, lo), jnp.where(ge, hi, mid)

        t, _hi = lax.fori_loop(
            0, 30, citer,
            (jnp.zeros((_L,), jnp.int32),
             jnp.full((_L,), 0x3F000001, jnp.int32)))

        def p3(v, c):
            n, z = c
            b = bits_v[pl.ds(v * _L, _L)]
            m = b > t
            n = n + plsc.all_reduce_population_count(m)
            z = z + jnp.where(m, lax.bitcast_convert_type(b, jnp.float32), 0.0)
            return n, z

        n_gt, z_vec = lax.fori_loop(
            0, _NV, p3,
            (jnp.zeros((_L,), jnp.int32), jnp.zeros((_L,), jnp.float32)))
        need = K_TOP - n_gt
        t_f = lax.bitcast_convert_type(t, jnp.float32)
        z_sum = jnp.broadcast_to(jnp.sum(z_vec), (_L,))
        z = z_sum + need.astype(jnp.float32) * t_f
        inv = 1.0 / (z + 1e-8)

        def p4(v, carry_eq):
            sl = pl.ds(v * _L, _L)
            b = bits_v[sl]
            s = row_v[sl]
            gt = b > t
            eq = b == t
            eqi = eq.astype(jnp.int32)
            rank = carry_eq + (plsc.cumsum(eqi) - eqi)
            mask = gt | (eq & (rank < need))
            act_v[sl] = jnp.where(mask, s * inv, 0.0)
            return carry_eq + plsc.all_reduce_population_count(eq)

        lax.fori_loop(0, _NV, p4, jnp.zeros((_L,), jnp.int32))
        pltpu.sync_copy(act_v, out_hbm.at[r])


@functools.cache
def _make_sc_select():
    return pl.kernel(
        _sc_select_body,
        mesh=plsc.VectorSubcoreMesh(core_axis_name="c", subcore_axis_name="s"),
        out_type=jax.ShapeDtypeStruct((B, A), jnp.float32),
        scratch_types=[
            pltpu.VMEM((A,), jnp.float32),
            pltpu.VMEM((A,), jnp.int32),
            pltpu.VMEM((A,), jnp.float32),
        ],
        compiler_params=pltpu.CompilerParams(needs_layout_passes=False),
    )


@jax.jit
def kernel(signal_features, W, b):
    scores = pl.pallas_call(
        _score_body,
        grid=(B // B_TILE,),
        in_specs=[
            pl.BlockSpec((B_TILE, A, D), lambda i: (i, 0, 0)),
            pl.BlockSpec((D, KM1), lambda i: (0, 0)),
            pl.BlockSpec((KM1, 1), lambda i: (0, 0)),
        ],
        out_specs=pl.BlockSpec((B_TILE, 1, A), lambda i: (i, 0, 0)),
        out_shape=jax.ShapeDtypeStruct((B, 1, A), jnp.float32),
    )(signal_features, W, b.reshape(KM1, 1))
    return _make_sc_select()(scores)
